# speedup vs baseline: 2.6043x; 2.6043x over previous
import functools

import jax
import jax.numpy as jnp
from jax import lax
from jax.experimental import pallas as pl
from jax.experimental.pallas import tpu as pltpu
from jax.experimental.pallas import tpu_sc as plsc

N = 10000
E = 320000
D = 128
L = 10

NC = 2
NS = 16
NW = NC * NS
CHUNK = 128
NCHUNKS = 80
EPT = CHUNK * NCHUNKS
E_PAD = EPT * NW
N_PAD = 10240
SLAB = N_PAD // NS
LAST = N - (NS - 1) * SLAB

_MESH = plsc.VectorSubcoreMesh(
    core_axis_name="c", subcore_axis_name="s", num_cores=NC, num_subcores=NS
)


def _fill(buf, val):
    vec = jnp.full((16,), val, jnp.float32)

    def body(r, _):
        for k in range(D // 16):
            buf[r, pl.ds(k * 16, 16)] = vec
        return 0

    lax.fori_loop(0, CHUNK, body, 0)


def _zero_slab(rows_v, acc_sh, s):
    _fill(rows_v, 0.0)
    slab = pl.multiple_of(s * SLAB, CHUNK)
    for k in range(SLAB // CHUNK):
        pltpu.sync_copy(rows_v, acc_sh.at[pl.ds(slab + k * CHUNK, CHUNK)])


def _copy_out(acc_sh, out_hbm, c, s):
    start = pl.multiple_of(s * SLAB, CHUNK)

    @pl.when(s < NS - 1)
    def _():
        pltpu.sync_copy(acc_sh.at[pl.ds(start, SLAB)],
                        out_hbm.at[c, pl.ds(start, SLAB)])

    @pl.when(s == NS - 1)
    def _():
        pltpu.sync_copy(acc_sh.at[pl.ds(start, LAST)],
                        out_hbm.at[c, pl.ds(start, LAST)])


@functools.partial(
    pl.kernel,
    out_type=jax.ShapeDtypeStruct((NC, N, D), jnp.float32),
    mesh=_MESH,
    scratch_types=[
        pltpu.VMEM((CHUNK,), jnp.int32),
        pltpu.VMEM((CHUNK,), jnp.int32),
        pltpu.VMEM((CHUNK, D), jnp.float32),
        pltpu.VMEM_SHARED((N_PAD, D), jnp.float32),
        pltpu.SemaphoreType.DMA,
    ],
)
def _sc_agg(h_hbm, src_hbm, dst_hbm, out_hbm, src_v, dst_v, rows_v, acc_sh, sem):
    c = lax.axis_index("c")
    s = lax.axis_index("s")
    wid = s * NC + c

    _zero_slab(rows_v, acc_sh, s)
    plsc.subcore_barrier()

    ebase = pl.multiple_of(wid * EPT, CHUNK)

    def chunk(ci, _):
        base = pl.multiple_of(ebase + ci * CHUNK, CHUNK)
        pltpu.sync_copy(src_hbm.at[pl.ds(base, CHUNK)], src_v)
        pltpu.sync_copy(dst_hbm.at[pl.ds(base, CHUNK)], dst_v)
        pltpu.async_copy(h_hbm.at[src_v], rows_v, sem).wait()
        pltpu.sync_copy(rows_v, acc_sh.at[dst_v], add=True)
        return 0

    lax.fori_loop(0, NCHUNKS, chunk, 0)
    plsc.subcore_barrier()
    _copy_out(acc_sh, out_hbm, c, s)


@functools.partial(
    pl.kernel,
    out_type=jax.ShapeDtypeStruct((NC, N, D), jnp.float32),
    mesh=_MESH,
    scratch_types=[
        pltpu.VMEM((CHUNK,), jnp.int32),
        pltpu.VMEM((CHUNK, D), jnp.float32),
        pltpu.VMEM_SHARED((N_PAD, D), jnp.float32),
        pltpu.SemaphoreType.DMA,
    ],
)
def _sc_deg(dst_hbm, out_hbm, dst_v, ones_v, acc_sh, sem):
    c = lax.axis_index("c")
    s = lax.axis_index("s")
    wid = s * NC + c

    _zero_slab(ones_v, acc_sh, s)
    _fill(ones_v, 1.0)
    plsc.subcore_barrier()

    ebase = pl.multiple_of(wid * EPT, CHUNK)

    def chunk(ci, _):
        base = pl.multiple_of(ebase + ci * CHUNK, CHUNK)
        pltpu.sync_copy(dst_hbm.at[pl.ds(base, CHUNK)], dst_v)
        pltpu.sync_copy(ones_v, acc_sh.at[dst_v], add=True)
        return 0

    lax.fori_loop(0, NCHUNKS, chunk, 0)
    plsc.subcore_barrier()
    _copy_out(acc_sh, out_hbm, c, s)


ROWS_BLK = 2000


def _invdeg_body(dp_ref, o_ref):
    deg = dp_ref[0] + dp_ref[1]
    o_ref[...] = 1.0 / jnp.maximum(deg, 1.0)


def _tc_invdeg(deg_p):
    return pl.pallas_call(
        _invdeg_body,
        grid=(N // ROWS_BLK,),
        in_specs=[pl.BlockSpec((NC, ROWS_BLK, D), lambda i: (0, i, 0))],
        out_specs=pl.BlockSpec((ROWS_BLK, D), lambda i: (i, 0)),
        out_shape=jax.ShapeDtypeStruct((N, D), jnp.float32),
    )(deg_p)


def _layer_body(relu, p_ref, h_ref, inv_ref, wl_ref, wr_ref, b_ref, o_ref):
    agg = (p_ref[0] + p_ref[1]) * inv_ref[...]
    dn = (((1,), (1,)), ((), ()))
    acc = lax.dot_general(agg, wl_ref[...], dn, preferred_element_type=jnp.float32)
    acc = acc + lax.dot_general(h_ref[...], wr_ref[...], dn,
                                preferred_element_type=jnp.float32)
    acc = acc + b_ref[...]
    o_ref[...] = jnp.maximum(acc, 0.0) if relu else acc


def _tc_layer(p, h, invd, wl, wr, bb, relu):
    return pl.pallas_call(
        functools.partial(_layer_body, relu),
        grid=(N // ROWS_BLK,),
        in_specs=[
            pl.BlockSpec((NC, ROWS_BLK, D), lambda i: (0, i, 0)),
            pl.BlockSpec((ROWS_BLK, D), lambda i: (i, 0)),
            pl.BlockSpec((ROWS_BLK, D), lambda i: (i, 0)),
            pl.BlockSpec((D, D), lambda i: (0, 0)),
            pl.BlockSpec((D, D), lambda i: (0, 0)),
            pl.BlockSpec((1, D), lambda i: (0, 0)),
        ],
        out_specs=pl.BlockSpec((ROWS_BLK, D), lambda i: (i, 0)),
        out_shape=jax.ShapeDtypeStruct((N, D), jnp.float32),
    )(p, h, invd, wl, wr, bb)


def kernel(x, edge_index, Wl, Wr, b):
    src = edge_index[0].astype(jnp.int32)
    dst = edge_index[1].astype(jnp.int32)
    pad = E_PAD - E
    src_p = jnp.concatenate([src, jnp.zeros((pad,), jnp.int32)])
    dst_p = jnp.concatenate([dst, jnp.full((pad,), N, jnp.int32)])

    deg_p = _sc_deg(dst_p)
    invd = _tc_invdeg(deg_p)

    h = x
    for i in range(L):
        p = _sc_agg(h, src_p, dst_p)
        h = _tc_layer(p, h, invd, Wl[i], Wr[i], b[i][None, :], relu=(i < L - 1))
    return h


# speedup vs baseline: 3.2146x; 1.2344x over previous
import functools

import jax
import jax.numpy as jnp
from jax import lax
from jax.experimental import pallas as pl
from jax.experimental.pallas import tpu as pltpu
from jax.experimental.pallas import tpu_sc as plsc

N = 10000
E = 320000
D = 128
L = 10

NC = 2
NS = 16
NW = NC * NS
CHUNK = 128
NCHUNKS = 80
EPT = CHUNK * NCHUNKS
E_PAD = EPT * NW
N_PAD = 10240
SLAB = N_PAD // NS
LAST = N - (NS - 1) * SLAB
NBUF = 2
DEG_W = D
NPH = 4
GCH = NCHUNKS // NPH

_MESH = plsc.VectorSubcoreMesh(
    core_axis_name="c", subcore_axis_name="s", num_cores=NC, num_subcores=NS
)


def _fill(buf, val, width=D):
    vec = jnp.full((16,), val, jnp.float32)

    def body(r, _):
        for k in range(width // 16):
            buf[r, pl.ds(k * 16, 16)] = vec
        return 0

    lax.fori_loop(0, CHUNK, body, 0)


def _zero_slab(zbuf, acc_sh, s, width=D):
    _fill(zbuf, 0.0, width)
    slab = pl.multiple_of(s * SLAB, CHUNK)
    for k in range(SLAB // CHUNK):
        pltpu.sync_copy(zbuf, acc_sh.at[pl.ds(slab + k * CHUNK, CHUNK)])


def _copy_out(acc_sh, out_hbm, c, s):
    start = pl.multiple_of(s * SLAB, CHUNK)

    @pl.when(s < NS - 1)
    def _():
        pltpu.sync_copy(acc_sh.at[pl.ds(start, SLAB)],
                        out_hbm.at[c, pl.ds(start, SLAB)])

    @pl.when(s == NS - 1)
    def _():
        pltpu.sync_copy(acc_sh.at[pl.ds(start, LAST)],
                        out_hbm.at[c, pl.ds(start, LAST)])


@functools.partial(
    pl.kernel,
    out_type=jax.ShapeDtypeStruct((NC, N, D), jnp.float32),
    mesh=_MESH,
    scratch_types=[
        pltpu.VMEM((CHUNK, D), jnp.float32),
        pltpu.VMEM((CHUNK, D), jnp.float32),
        pltpu.VMEM((CHUNK,), jnp.int32),
        pltpu.VMEM((CHUNK,), jnp.int32),
        pltpu.VMEM((CHUNK,), jnp.int32),
        pltpu.VMEM((CHUNK,), jnp.int32),
        pltpu.VMEM_SHARED((N_PAD, D), jnp.float32),
        [pltpu.SemaphoreType.DMA] * 2,
        [pltpu.SemaphoreType.DMA] * 2,
        [pltpu.SemaphoreType.DMA] * 2,
        [pltpu.SemaphoreType.DMA] * 2,
    ],
)
def _sc_agg(h_hbm, idx_hbm, out_hbm,
            rows0, rows1, src0, src1, dst0, dst1, acc_sh,
            xsems, dsems, gsems, ssems):
    c = lax.axis_index("c")
    s = lax.axis_index("s")
    wid = s * NC + c
    rows = (rows0, rows1)
    srcb = (src0, src1)
    dstb = (dst0, dst1)

    for b in range(2):
        pltpu.async_copy(idx_hbm.at[wid, b, 0], srcb[b], xsems[b])
        pltpu.async_copy(idx_hbm.at[wid, b, 1], dstb[b], dsems[b])
    _zero_slab(rows0, acc_sh, s)
    plsc.subcore_barrier()

    for b in range(2):
        pltpu.make_async_copy(idx_hbm.at[wid, 0, 0], srcb[b], xsems[b]).wait()
        pltpu.async_copy(h_hbm.at[srcb[b]], rows[b], gsems[b])
    for b in range(2):
        pltpu.make_async_copy(h_hbm.at[srcb[b]], rows[b], gsems[b]).wait()
        pltpu.async_copy(idx_hbm.at[wid, b + 2, 0], srcb[b], xsems[b])
        pltpu.make_async_copy(idx_hbm.at[wid, 0, 1], dstb[b], dsems[b]).wait()
        pltpu.async_copy(rows[b], acc_sh.at[dstb[b]], ssems[b], add=True)

    def group(g, _):
        for b in range(2):
            ci = 2 * g + b
            pltpu.make_async_copy(rows[b], acc_sh.at[dstb[b]],
                                  ssems[b]).wait()
            pltpu.async_copy(idx_hbm.at[wid, ci, 1], dstb[b], dsems[b])
            pltpu.make_async_copy(idx_hbm.at[wid, 0, 0], srcb[b],
                                  xsems[b]).wait()
            pltpu.async_copy(h_hbm.at[srcb[b]], rows[b], gsems[b])
        for b in range(2):
            ci = 2 * g + b
            pltpu.make_async_copy(h_hbm.at[srcb[b]], rows[b],
                                  gsems[b]).wait()
            pltpu.async_copy(idx_hbm.at[wid, ci + 2, 0], srcb[b], xsems[b])
            pltpu.make_async_copy(idx_hbm.at[wid, 0, 1], dstb[b],
                                  dsems[b]).wait()
            pltpu.async_copy(rows[b], acc_sh.at[dstb[b]], ssems[b], add=True)
        return 0

    lax.fori_loop(1, NCHUNKS // 2 - 1, group, 0)

    for b in range(2):
        ci = NCHUNKS - 2 + b
        pltpu.make_async_copy(rows[b], acc_sh.at[dstb[b]], ssems[b]).wait()
        pltpu.async_copy(idx_hbm.at[wid, ci, 1], dstb[b], dsems[b])
        pltpu.make_async_copy(idx_hbm.at[wid, 0, 0], srcb[b],
                              xsems[b]).wait()
        pltpu.async_copy(h_hbm.at[srcb[b]], rows[b], gsems[b])
    for b in range(2):
        pltpu.make_async_copy(h_hbm.at[srcb[b]], rows[b], gsems[b]).wait()
        pltpu.make_async_copy(idx_hbm.at[wid, 0, 1], dstb[b],
                              dsems[b]).wait()
        pltpu.async_copy(rows[b], acc_sh.at[dstb[b]], ssems[b], add=True)
    for b in range(2):
        pltpu.make_async_copy(rows[b], acc_sh.at[dstb[b]], ssems[b]).wait()

    plsc.subcore_barrier()
    _copy_out(acc_sh, out_hbm, c, s)


@functools.partial(
    pl.kernel,
    out_type=jax.ShapeDtypeStruct((NC, N, DEG_W), jnp.float32),
    mesh=_MESH,
    scratch_types=[
        pltpu.VMEM((CHUNK, DEG_W), jnp.float32),
        pltpu.VMEM((CHUNK,), jnp.int32),
        pltpu.VMEM_SHARED((N_PAD, DEG_W), jnp.float32),
        pltpu.SemaphoreType.DMA,
    ],
)
def _sc_deg(idx_hbm, out_hbm, ones_v, dst0, acc_sh, sem):
    c = lax.axis_index("c")
    s = lax.axis_index("s")
    wid = s * NC + c

    _zero_slab(ones_v, acc_sh, s, DEG_W)
    _fill(ones_v, 1.0, DEG_W)
    plsc.subcore_barrier()

    def chunk(ci, _):
        pltpu.sync_copy(idx_hbm.at[wid, ci, 1], dst0)
        pltpu.sync_copy(ones_v, acc_sh.at[dst0], add=True)
        return 0

    lax.fori_loop(0, NCHUNKS, chunk, 0)
    plsc.subcore_barrier()
    _copy_out(acc_sh, out_hbm, c, s)


ROWS_BLK = 2000


def _invdeg_body(dp_ref, o_ref):
    deg = dp_ref[0, :, :1] + dp_ref[1, :, :1]
    o_ref[...] = jnp.broadcast_to(1.0 / jnp.maximum(deg, 1.0), (ROWS_BLK, D))


def _tc_invdeg(deg_p):
    return pl.pallas_call(
        _invdeg_body,
        grid=(N // ROWS_BLK,),
        in_specs=[pl.BlockSpec((NC, ROWS_BLK, DEG_W), lambda i: (0, i, 0))],
        out_specs=pl.BlockSpec((ROWS_BLK, D), lambda i: (i, 0)),
        out_shape=jax.ShapeDtypeStruct((N, D), jnp.float32),
    )(deg_p)


def _layer_body(relu, p_ref, h_ref, inv_ref, wl_ref, wr_ref, b_ref, o_ref):
    agg = (p_ref[0] + p_ref[1]) * inv_ref[...]
    dn = (((1,), (1,)), ((), ()))
    acc = lax.dot_general(agg, wl_ref[...], dn, preferred_element_type=jnp.float32)
    acc = acc + lax.dot_general(h_ref[...], wr_ref[...], dn,
                                preferred_element_type=jnp.float32)
    acc = acc + b_ref[...]
    o_ref[...] = jnp.maximum(acc, 0.0) if relu else acc


def _tc_layer(p, h, invd, wl, wr, bb, relu):
    return pl.pallas_call(
        functools.partial(_layer_body, relu),
        grid=(N // ROWS_BLK,),
        in_specs=[
            pl.BlockSpec((NC, ROWS_BLK, D), lambda i: (0, i, 0)),
            pl.BlockSpec((ROWS_BLK, D), lambda i: (i, 0)),
            pl.BlockSpec((ROWS_BLK, D), lambda i: (i, 0)),
            pl.BlockSpec((D, D), lambda i: (0, 0)),
            pl.BlockSpec((D, D), lambda i: (0, 0)),
            pl.BlockSpec((1, D), lambda i: (0, 0)),
        ],
        out_specs=pl.BlockSpec((ROWS_BLK, D), lambda i: (i, 0)),
        out_shape=jax.ShapeDtypeStruct((N, D), jnp.float32),
    )(p, h, invd, wl, wr, bb)


def kernel(x, edge_index, Wl, Wr, b):
    src = edge_index[0].astype(jnp.int32)
    dst = edge_index[1].astype(jnp.int32)
    pad = E_PAD - E
    src_p = jnp.concatenate([src, jnp.zeros((pad,), jnp.int32)])
    dst_p = jnp.concatenate([dst, jnp.full((pad,), N, jnp.int32)])
    idx = jnp.stack([src_p.reshape(NW, NCHUNKS, CHUNK),
                     dst_p.reshape(NW, NCHUNKS, CHUNK)], axis=2)

    deg_p = _sc_deg(idx)
    invd = _tc_invdeg(deg_p)

    h = x
    for i in range(L):
        p = _sc_agg(h, idx)
        h = _tc_layer(p, h, invd, Wl[i], Wr[i], b[i][None, :], relu=(i < L - 1))
    return h
